# baseline (speedup 1.0000x reference)
import jax
import jax.numpy as jnp
from jax.experimental import pallas as pl
from jax.experimental.pallas import tpu as pltpu

_SHIFT = 8192
_NUM_DMAS = 8


def _copy_body(in_ref, out_ref, sems):
    rows, out_len = out_ref.shape
    chunk = rows // _NUM_DMAS
    for i in range(_NUM_DMAS):
        pltpu.make_async_copy(
            in_ref.at[pl.ds(i * chunk, chunk), pl.ds(0, out_len)],
            out_ref.at[pl.ds(i * chunk, chunk), :],
            sems.at[i],
        ).start()
    for i in range(_NUM_DMAS):
        pltpu.make_async_copy(
            in_ref.at[pl.ds(i * chunk, chunk), pl.ds(0, out_len)],
            out_ref.at[pl.ds(i * chunk, chunk), :],
            sems.at[i],
        ).wait()


def kernel(wav):
    s, b, c, length = wav.shape
    out_len = length - _SHIFT
    rows = s * b * c
    x = wav.reshape(rows, length)

    out = pl.pallas_call(
        _copy_body,
        in_specs=[pl.BlockSpec(memory_space=pl.ANY)],
        out_specs=pl.BlockSpec(memory_space=pl.ANY),
        out_shape=jax.ShapeDtypeStruct((rows, out_len), wav.dtype),
        scratch_shapes=[pltpu.SemaphoreType.DMA((_NUM_DMAS,))],
    )(x)
    return out.reshape(s, b, c, out_len)


# speedup vs baseline: 8.6701x; 8.6701x over previous
import jax
import jax.numpy as jnp
from jax.experimental import pallas as pl
from jax.experimental.pallas import tpu as pltpu

_SHIFT = 8192


def _copy_body(in_ref, out_ref):
    out_ref[...] = in_ref[...]


def kernel(wav):
    s, b, c, length = wav.shape
    out_len = length - _SHIFT
    rows = s * b * c
    x = wav.reshape(rows, length)

    rows_per_block = 8
    out = pl.pallas_call(
        _copy_body,
        grid=(rows // rows_per_block,),
        in_specs=[pl.BlockSpec((rows_per_block, out_len), lambda i: (i, 0))],
        out_specs=pl.BlockSpec((rows_per_block, out_len), lambda i: (i, 0)),
        out_shape=jax.ShapeDtypeStruct((rows, out_len), wav.dtype),
        compiler_params=pltpu.CompilerParams(
            dimension_semantics=("parallel",),
        ),
    )(x)
    return out.reshape(s, b, c, out_len)
